# baseline (speedup 1.0000x reference)
import jax
import jax.numpy as jnp
from jax import lax
from jax.experimental import pallas as pl
from jax.experimental.pallas import tpu as pltpu
from jax.experimental.pallas import tpu_sc as plsc

N_ROWS = 128
N_COLS = 32768
L = 16
NC = 2
NS = 16
NW = NC * NS
ROWS_PER_W = N_ROWS // NW
CHUNKS = N_COLS // L

_mesh = plsc.VectorSubcoreMesh(core_axis_name="c", subcore_axis_name="s",
                               num_cores=NC, num_subcores=NS)

_SCRATCH = [
    pltpu.VMEM((N_COLS,), jnp.float32),
    pltpu.VMEM((N_COLS,), jnp.float32),
    pltpu.VMEM((L,), jnp.int32),
    pltpu.VMEM((L,), jnp.float32),
    pltpu.VMEM((L,), jnp.int32),
    pltpu.SemaphoreType.DMA,
    pltpu.SemaphoreType.DMA,
]


def _argmax_body(x_hbm, out_hbm, buf0, buf1, resv, tmpv, tmpi, sem0, sem1):
    c = lax.axis_index("c")
    s = lax.axis_index("s")
    w = c * NS + s
    row0 = w * ROWS_PER_W
    iota = lax.iota(jnp.int32, L)

    bufs = (buf0, buf1)
    sems = (sem0, sem1)
    descs = [None, None]

    def start_row(r):
        descs[r % 2] = pltpu.async_copy(
            x_hbm.at[pl.ds((row0 + r) * N_COLS, N_COLS)], bufs[r % 2],
            sems[r % 2])

    start_row(0)
    results = jnp.zeros((L,), jnp.int32)
    for r in range(ROWS_PER_W):
        if r + 1 < ROWS_PER_W:
            start_row(r + 1)
        descs[r % 2].wait()
        cur = bufs[r % 2]

        def chunk_body(i, carry, cur=cur):
            best, bidx = carry
            v = cur[pl.ds(i * L, L)]
            m = v > best
            best = jnp.where(m, v, best)
            bidx = jnp.where(m, jnp.full((L,), i, jnp.int32), bidx)
            return best, bidx

        best, bidx = lax.fori_loop(
            0, CHUNKS, chunk_body,
            (jnp.full((L,), -jnp.inf, jnp.float32), jnp.zeros((L,), jnp.int32)),
            unroll=8)

        idxv = bidx * L + iota
        v, ix = best, idxv
        for k in (8, 4, 2, 1):
            tmpv[...] = v
            tmpi[...] = ix
            perm = jnp.bitwise_xor(iota, k)
            v2 = plsc.load_gather(tmpv, [perm])
            i2 = plsc.load_gather(tmpi, [perm])
            m = (v2 > v) | ((v2 == v) & (i2 < ix))
            v = jnp.where(m, v2, v)
            ix = jnp.where(m, i2, ix)
        results = jnp.where(iota == r, ix, results)

    resv[...] = results
    pltpu.sync_copy(resv, out_hbm.at[w])


_argmax_sc = pl.kernel(
    _argmax_body,
    out_type=jax.ShapeDtypeStruct((NW, L), jnp.int32),
    mesh=_mesh,
    compiler_params=pltpu.CompilerParams(needs_layout_passes=False),
    scratch_types=_SCRATCH,
)


def kernel(x):
    board = _argmax_sc(jnp.reshape(x, (-1,)))
    return board[:, :ROWS_PER_W].reshape(N_ROWS)


# speedup vs baseline: 1.0316x; 1.0316x over previous
import jax
import jax.numpy as jnp
from jax import lax
from jax.experimental import pallas as pl
from jax.experimental.pallas import tpu as pltpu
from jax.experimental.pallas import tpu_sc as plsc

N_ROWS = 128
N_COLS = 32768
L = 16
NC = 2
NS = 16
NW = NC * NS
ROWS_PER_W = N_ROWS // NW
CHUNKS = N_COLS // L
BLK_CHUNKS = 64
NBLK = CHUNKS // BLK_CHUNKS
ACC = 4
IMAX = jnp.iinfo(jnp.int32).max

_mesh = plsc.VectorSubcoreMesh(core_axis_name="c", subcore_axis_name="s",
                               num_cores=NC, num_subcores=NS)

_SCRATCH = [
    pltpu.VMEM((N_COLS,), jnp.float32),
    pltpu.VMEM((N_COLS,), jnp.float32),
    pltpu.VMEM((NBLK * L,), jnp.float32),
    pltpu.VMEM((L,), jnp.int32),
    pltpu.VMEM((L,), jnp.float32),
    pltpu.VMEM((L,), jnp.int32),
    pltpu.SemaphoreType.DMA,
    pltpu.SemaphoreType.DMA,
]


def _argmax_body(x_hbm, out_hbm, buf0, buf1, blkmax, resv, tmpv, tmpi,
                 sem0, sem1):
    c = lax.axis_index("c")
    s = lax.axis_index("s")
    w = c * NS + s
    row0 = w * ROWS_PER_W
    iota = lax.iota(jnp.int32, L)
    neg_inf = jnp.full((L,), -jnp.inf, jnp.float32)
    imax_v = jnp.full((L,), IMAX, jnp.int32)

    def bfly_max(v):
        for k in (8, 4, 2, 1):
            tmpv[...] = v
            v = jnp.maximum(v, plsc.load_gather(tmpv, [iota ^ k]))
        return v

    def bfly_min_i32(ix):
        for k in (8, 4, 2, 1):
            tmpi[...] = ix
            ix = jnp.minimum(ix, plsc.load_gather(tmpi, [iota ^ k]))
        return ix

    def bfly_argmax(v, ix):
        for k in (8, 4, 2, 1):
            tmpv[...] = v
            tmpi[...] = ix
            v2 = plsc.load_gather(tmpv, [iota ^ k])
            i2 = plsc.load_gather(tmpi, [iota ^ k])
            m = (v2 > v) | ((v2 == v) & (i2 < ix))
            v = jnp.where(m, v2, v)
            ix = jnp.where(m, i2, ix)
        return v, ix

    bufs = (buf0, buf1)
    sems = (sem0, sem1)
    descs = [None, None]

    def start_row(r):
        descs[r % 2] = pltpu.async_copy(
            x_hbm.at[pl.ds((row0 + r) * N_COLS, N_COLS)], bufs[r % 2],
            sems[r % 2])

    start_row(0)
    results = jnp.zeros((L,), jnp.int32)
    for r in range(ROWS_PER_W):
        if r + 1 < ROWS_PER_W:
            start_row(r + 1)
        descs[r % 2].wait()
        cur = bufs[r % 2]

        @plsc.parallel_loop(0, NBLK)
        def _p1(b):
            base = b * (BLK_CHUNKS * L)

            @plsc.parallel_loop(0, BLK_CHUNKS, step=ACC, unroll=4,
                                carry=(neg_inf,) * ACC)
            def accs(i, ms):
                return tuple(
                    jnp.maximum(m, cur[pl.ds(base + (i + a) * L, L)])
                    for a, m in enumerate(ms))

            bm = jnp.maximum(jnp.maximum(accs[0], accs[1]),
                             jnp.maximum(accs[2], accs[3]))
            blkmax[pl.ds(b * L, L)] = bm

        @plsc.parallel_loop(0, NBLK, unroll=4, carry=neg_inf)
        def gm(i, m):
            return jnp.maximum(m, blkmax[pl.ds(i * L, L)])

        mx = bfly_max(gm)

        @plsc.parallel_loop(0, NBLK, unroll=4, carry=imax_v)
        def firstb(i, fb):
            v = blkmax[pl.ds(i * L, L)]
            return jnp.minimum(fb, jnp.where(v == mx,
                                             jnp.full((L,), i, jnp.int32),
                                             imax_v))

        bstar = bfly_min_i32(firstb)[0]

        base = bstar * (BLK_CHUNKS * L)

        @plsc.parallel_loop(0, BLK_CHUNKS, unroll=2,
                            carry=(neg_inf, jnp.zeros((L,), jnp.int32)))
        def scan(i, cr):
            best, bidx = cr
            v = cur[pl.ds(base + i * L, L)]
            m = v > best
            return (jnp.where(m, v, best),
                    jnp.where(m, jnp.full((L,), i, jnp.int32), bidx))

        best, bidx = scan
        idxv = (bstar * BLK_CHUNKS + bidx) * L + iota
        _, ix = bfly_argmax(best, idxv)
        results = jnp.where(iota == r, ix, results)

    resv[...] = results
    pltpu.sync_copy(resv, out_hbm.at[w])


_argmax_sc = pl.kernel(
    _argmax_body,
    out_type=jax.ShapeDtypeStruct((NW, L), jnp.int32),
    mesh=_mesh,
    compiler_params=pltpu.CompilerParams(needs_layout_passes=False),
    scratch_types=_SCRATCH,
)


def kernel(x):
    board = _argmax_sc(jnp.reshape(x, (-1,)))
    return board[:, :ROWS_PER_W].reshape(N_ROWS)


# speedup vs baseline: 1.5612x; 1.5134x over previous
import jax
import jax.numpy as jnp
from jax import lax
from jax.experimental import pallas as pl
from jax.experimental.pallas import tpu as pltpu
from jax.experimental.pallas import tpu_sc as plsc

N_ROWS = 128
N_COLS = 32768
L = 16
NC = 2
NS = 16
NW = NC * NS
ROWS_PER_W = N_ROWS // NW
CHUNKS = N_COLS // L
BLK_CHUNKS = 64
NBLK = CHUNKS // BLK_CHUNKS
ACC = 4
IMAX = jnp.iinfo(jnp.int32).max

_mesh = plsc.VectorSubcoreMesh(core_axis_name="c", subcore_axis_name="s",
                               num_cores=NC, num_subcores=NS)

_SCRATCH = [
    pltpu.VMEM((N_COLS,), jnp.float32),
    pltpu.VMEM((N_COLS,), jnp.float32),
    pltpu.VMEM((NBLK * L,), jnp.float32),
    pltpu.VMEM((L,), jnp.int32),
    pltpu.VMEM((L,), jnp.float32),
    pltpu.VMEM((L,), jnp.int32),
    pltpu.SemaphoreType.DMA,
    pltpu.SemaphoreType.DMA,
]


def _argmax_body(x_hbm, out_hbm, buf0, buf1, blkmax, resv, tmpv, tmpi,
                 sem0, sem1):
    c = lax.axis_index("c")
    s = lax.axis_index("s")
    w = c * NS + s
    row0 = w * ROWS_PER_W
    iota = lax.iota(jnp.int32, L)
    neg_inf = jnp.full((L,), -jnp.inf, jnp.float32)
    imax_v = jnp.full((L,), IMAX, jnp.int32)

    def bfly_max(v):
        for k in (8, 4, 2, 1):
            tmpv[...] = v
            v = jnp.maximum(v, plsc.load_gather(tmpv, [iota ^ k]))
        return v

    def bfly_min_i32(ix):
        for k in (8, 4, 2, 1):
            tmpi[...] = ix
            ix = jnp.minimum(ix, plsc.load_gather(tmpi, [iota ^ k]))
        return ix

    def bfly_argmax(v, ix):
        for k in (8, 4, 2, 1):
            tmpv[...] = v
            tmpi[...] = ix
            v2 = plsc.load_gather(tmpv, [iota ^ k])
            i2 = plsc.load_gather(tmpi, [iota ^ k])
            m = (v2 > v) | ((v2 == v) & (i2 < ix))
            v = jnp.where(m, v2, v)
            ix = jnp.where(m, i2, ix)
        return v, ix

    bufs = (buf0, buf1)
    sems = (sem0, sem1)
    descs = [None, None]

    def start_row(r):
        descs[r % 2] = pltpu.async_copy(
            x_hbm.at[row0 + r], bufs[r % 2], sems[r % 2])

    start_row(0)
    results = jnp.zeros((L,), jnp.int32)
    for r in range(ROWS_PER_W):
        if r + 1 < ROWS_PER_W:
            start_row(r + 1)
        descs[r % 2].wait()
        cur = bufs[r % 2]

        @plsc.parallel_loop(0, NBLK)
        def _p1(b):
            base = b * (BLK_CHUNKS * L)

            @plsc.parallel_loop(0, BLK_CHUNKS, step=ACC, unroll=4,
                                carry=(neg_inf,) * ACC)
            def accs(i, ms):
                return tuple(
                    jnp.maximum(m, cur[pl.ds(base + (i + a) * L, L)])
                    for a, m in enumerate(ms))

            bm = jnp.maximum(jnp.maximum(accs[0], accs[1]),
                             jnp.maximum(accs[2], accs[3]))
            blkmax[pl.ds(b * L, L)] = bm

        @plsc.parallel_loop(0, NBLK, unroll=4, carry=neg_inf)
        def gm(i, m):
            return jnp.maximum(m, blkmax[pl.ds(i * L, L)])

        mx = bfly_max(gm)

        @plsc.parallel_loop(0, NBLK, unroll=4, carry=imax_v)
        def firstb(i, fb):
            v = blkmax[pl.ds(i * L, L)]
            return jnp.minimum(fb, jnp.where(v == mx,
                                             jnp.full((L,), i, jnp.int32),
                                             imax_v))

        bstar = bfly_min_i32(firstb)[0]

        base = bstar * (BLK_CHUNKS * L)

        @plsc.parallel_loop(0, BLK_CHUNKS, unroll=2,
                            carry=(neg_inf, jnp.zeros((L,), jnp.int32)))
        def scan(i, cr):
            best, bidx = cr
            v = cur[pl.ds(base + i * L, L)]
            m = v > best
            return (jnp.where(m, v, best),
                    jnp.where(m, jnp.full((L,), i, jnp.int32), bidx))

        best, bidx = scan
        idxv = (bstar * BLK_CHUNKS + bidx) * L + iota
        _, ix = bfly_argmax(best, idxv)
        results = jnp.where(iota == r, ix, results)

    resv[...] = results
    pltpu.sync_copy(resv, out_hbm.at[w])


_argmax_sc = pl.kernel(
    _argmax_body,
    out_type=jax.ShapeDtypeStruct((NW, L), jnp.int32),
    mesh=_mesh,
    compiler_params=pltpu.CompilerParams(needs_layout_passes=False),
    scratch_types=_SCRATCH,
)


def kernel(x):
    board = _argmax_sc(x)
    return board[:, :ROWS_PER_W].reshape(N_ROWS)
